# baseline (speedup 1.0000x reference)
import functools

import jax
import jax.numpy as jnp
from jax import lax
from jax.experimental import pallas as pl
from jax.experimental.pallas import tpu as pltpu
from jax.experimental.pallas import tpu_sc as plsc

NC = 2
NS = 16
NW = NC * NS
LANES = 16
CHUNK = 4096
VPC = CHUNK // LANES


def _sc_spmv_kernel(n, m, nnz):
    nnz_per_w = nnz // NW
    n_chunks = nnz_per_w // CHUNK
    assert nnz_per_w % CHUNK == 0 and n_chunks % 2 == 0

    mesh = plsc.VectorSubcoreMesh(core_axis_name="c", subcore_axis_name="s")
    f32 = jnp.float32
    i32 = jnp.int32

    @functools.partial(
        pl.kernel,
        out_type=(
            jax.ShapeDtypeStruct((NW, n), f32),
            jax.ShapeDtypeStruct((NW, n), f32),
            jax.ShapeDtypeStruct((NW, m), f32),
        ),
        mesh=mesh,
        compiler_params=pltpu.CompilerParams(needs_layout_passes=False),
        scratch_types=dict(
            x_v=pltpu.MemorySpace.VMEM((n,), f32),
            wy_v=pltpu.MemorySpace.VMEM((m,), f32),
            acc_px=pltpu.MemorySpace.VMEM((n,), f32),
            acc_aty=pltpu.MemorySpace.VMEM((n,), f32),
            acc_ax=pltpu.MemorySpace.VMEM((m,), f32),
            rows0=pltpu.MemorySpace.VMEM((CHUNK,), i32),
            cols0=pltpu.MemorySpace.VMEM((CHUNK,), i32),
            data0=pltpu.MemorySpace.VMEM((CHUNK,), f32),
            rows1=pltpu.MemorySpace.VMEM((CHUNK,), i32),
            cols1=pltpu.MemorySpace.VMEM((CHUNK,), i32),
            data1=pltpu.MemorySpace.VMEM((CHUNK,), f32),
            rows2=pltpu.MemorySpace.VMEM((CHUNK,), i32),
            cols2=pltpu.MemorySpace.VMEM((CHUNK,), i32),
            data2=pltpu.MemorySpace.VMEM((CHUNK,), f32),
            tmp_v=pltpu.MemorySpace.VMEM((CHUNK,), f32),
            semr0=pltpu.SemaphoreType.DMA,
            semc0=pltpu.SemaphoreType.DMA,
            semd0=pltpu.SemaphoreType.DMA,
            semr1=pltpu.SemaphoreType.DMA,
            semc1=pltpu.SemaphoreType.DMA,
            semd1=pltpu.SemaphoreType.DMA,
            semr2=pltpu.SemaphoreType.DMA,
            semc2=pltpu.SemaphoreType.DMA,
            semd2=pltpu.SemaphoreType.DMA,
        ),
    )
    def spmv(p_data, p_rows, p_cols, a_data, a_rows, a_cols, x_h, y_h, s_h,
             o_px, o_aty, o_ax,
             x_v, wy_v, acc_px, acc_aty, acc_ax,
             rows0, cols0, data0, rows1, cols1, data1,
             rows2, cols2, data2, tmp_v,
             semr0, semc0, semd0, semr1, semc1, semd1,
             semr2, semc2, semd2):
        cid = lax.axis_index("c")
        sid = lax.axis_index("s")
        wid = cid * NS + sid
        base = wid * nnz_per_w

        sets = ((rows0, cols0, data0, semr0, semc0, semd0),
                (rows1, cols1, data1, semr1, semc1, semd1),
                (rows2, cols2, data2, semr2, semc2, semd2))
        nbuf = len(sets)

        cx = pltpu.async_copy(x_h, x_v, semr1)
        for ch in range(m // CHUNK):
            off = ch * CHUNK
            cy = pltpu.async_copy(y_h.at[pl.ds(off, CHUNK)], data0, semr0)
            cs = pltpu.async_copy(s_h.at[pl.ds(off, CHUNK)], tmp_v, semc0)
            cy.wait()
            cs.wait()

            @plsc.parallel_loop(0, VPC, unroll=8)
            def _(i):
                sl = pl.ds(i * LANES, LANES)
                wy_v[pl.ds(off + i * LANES, LANES)] = jnp.maximum(
                    data0[sl] - tmp_v[sl], 0.0)

        @plsc.parallel_loop(0, n // LANES, unroll=8)
        def _(i):
            z = jnp.zeros((LANES,), f32)
            sl = pl.ds(i * LANES, LANES)
            acc_px[sl] = z
            acc_aty[sl] = z
            acc_ax[sl] = z

        cx.wait()

        def issue(buf, dh, rh, ch_, off):
            rows_v, cols_v, data_v, sr, sc, sd = buf
            c0 = pltpu.async_copy(rh.at[pl.ds(off, CHUNK)], rows_v, sr)
            c1 = pltpu.async_copy(ch_.at[pl.ds(off, CHUNK)], cols_v, sc)
            c2 = pltpu.async_copy(dh.at[pl.ds(off, CHUNK)], data_v, sd)
            return c0, c1, c2

        def wait(buf, dh, rh, ch_):
            rows_v, cols_v, data_v, sr, sc, sd = buf
            pltpu.make_async_copy(rh.at[pl.ds(0, CHUNK)], rows_v, sr).wait()
            pltpu.make_async_copy(ch_.at[pl.ds(0, CHUNK)], cols_v, sc).wait()
            pltpu.make_async_copy(dh.at[pl.ds(0, CHUNK)], data_v, sd).wait()

        def process_p(buf):
            rows_v, cols_v, data_v, *_ = buf

            @plsc.parallel_loop(0, VPC, unroll=16)
            def _(i):
                sl = pl.ds(i * LANES, LANES)
                vals = data_v[sl] * plsc.load_gather(x_v, [cols_v[sl]])
                plsc.addupdate_scatter(acc_px, [rows_v[sl]], vals)

        def process_a(buf):
            rows_v, cols_v, data_v, *_ = buf

            @plsc.parallel_loop(0, VPC, unroll=16)
            def _(i):
                sl = pl.ds(i * LANES, LANES)
                rows = rows_v[sl]
                cols = cols_v[sl]
                data = data_v[sl]
                plsc.addupdate_scatter(acc_ax, [rows],
                                       data * plsc.load_gather(x_v, [cols]))
                plsc.addupdate_scatter(acc_aty, [cols],
                                       data * plsc.load_gather(wy_v, [rows]))

        def pass_over(dh, rh, ch_, process):
            n_groups = -(-n_chunks // nbuf)
            for k in range(nbuf):
                issue(sets[k], dh, rh, ch_, base + k * CHUNK)

            def group_body(g, _):
                for k in range(nbuf):
                    ch = g * nbuf + k

                    @pl.when(ch < n_chunks)
                    def _():
                        wait(sets[k], dh, rh, ch_)
                        process(sets[k])

                    @pl.when(ch + nbuf < n_chunks)
                    def _():
                        issue(sets[k], dh, rh, ch_,
                              base + (ch + nbuf) * CHUNK)

                return 0

            lax.fori_loop(0, n_groups, group_body, 0)

        pass_over(p_data, p_rows, p_cols, process_p)
        pass_over(a_data, a_rows, a_cols, process_a)

        pltpu.sync_copy(acc_px, o_px.at[wid])
        pltpu.sync_copy(acc_aty, o_aty.at[wid])
        pltpu.sync_copy(acc_ax, o_ax.at[wid])

    return spmv


def _tc_combine(p_px, p_aty, p_ax, q2, b2, x2, y2, s2):
    n = q2.shape[1]
    m = b2.shape[1]
    f32 = jnp.float32

    def body(px_ref, aty_ref, ax_ref, q_ref, b_ref, x_ref, y_ref, s_ref,
             out_ref):
        px = jnp.sum(px_ref[...], axis=0, keepdims=True)
        aty = jnp.sum(aty_ref[...], axis=0, keepdims=True)
        ax = jnp.sum(ax_ref[...], axis=0, keepdims=True)
        q = q_ref[...]
        b = b_ref[...]
        x = x_ref[...]
        wy = jnp.maximum(y_ref[...] - s_ref[...], 0.0)
        out_ref[:, pl.ds(0, n)] = px + aty + q
        out_ref[:, pl.ds(n, m)] = b - ax
        xtpx = jnp.sum(x * px)
        qx = jnp.sum(q * x)
        bwy = jnp.sum(b * wy)
        out_ref[:, pl.ds(n + m, 1)] = jnp.reshape(
            -(qx + xtpx) - bwy + 0.5 * xtpx, (1, 1))

    return pl.pallas_call(
        body,
        out_shape=jax.ShapeDtypeStruct((1, n + m + 1), f32),
    )(p_px, p_aty, p_ax, q2, b2, x2, y2, s2)


def kernel(P_data, A_data, q, b, x, y, s, P_rows, P_cols, A_rows, A_cols):
    n = x.shape[0]
    m = y.shape[0]
    nnz = P_data.shape[0]

    spmv = _sc_spmv_kernel(n, m, nnz)
    p_px, p_aty, p_ax = spmv(P_data, P_rows, P_cols, A_data, A_rows, A_cols,
                             x, y, s)

    out = _tc_combine(
        p_px, p_aty, p_ax,
        q.reshape(1, n), b.reshape(1, m), x.reshape(1, n),
        y.reshape(1, m), s.reshape(1, m))

    return out.reshape(-1)


# speedup vs baseline: 1.0117x; 1.0117x over previous
import functools

import jax
import jax.numpy as jnp
from jax import lax
from jax.experimental import pallas as pl
from jax.experimental.pallas import tpu as pltpu
from jax.experimental.pallas import tpu_sc as plsc

NC = 2
NS = 16
NW = NC * NS
LANES = 16
CHUNK = 4096
VPC = CHUNK // LANES


def _sc_spmv_kernel(n, m, nnz):
    nnz_per_w = nnz // NW
    n_chunks = nnz_per_w // CHUNK
    assert nnz_per_w % CHUNK == 0 and n_chunks % 2 == 0
    assert n == m and n % CHUNK == 0

    mesh = plsc.VectorSubcoreMesh(core_axis_name="c", subcore_axis_name="s")
    f32 = jnp.float32
    i32 = jnp.int32

    @functools.partial(
        pl.kernel,
        out_type=(
            jax.ShapeDtypeStruct((NW, n), f32),
            jax.ShapeDtypeStruct((NW, n), f32),
            jax.ShapeDtypeStruct((NW, m), f32),
        ),
        mesh=mesh,
        compiler_params=pltpu.CompilerParams(needs_layout_passes=False),
        scratch_types=dict(
            x_v=pltpu.MemorySpace.VMEM((n,), f32),
            wy_v=pltpu.MemorySpace.VMEM((m,), f32),
            acc_px=pltpu.MemorySpace.VMEM((n,), f32),
            acc_aty=pltpu.MemorySpace.VMEM((n,), f32),
            acc_ax=pltpu.MemorySpace.VMEM((m,), f32),
            rows0=pltpu.MemorySpace.VMEM((CHUNK,), i32),
            cols0=pltpu.MemorySpace.VMEM((CHUNK,), i32),
            data0=pltpu.MemorySpace.VMEM((CHUNK,), f32),
            rows1=pltpu.MemorySpace.VMEM((CHUNK,), i32),
            cols1=pltpu.MemorySpace.VMEM((CHUNK,), i32),
            data1=pltpu.MemorySpace.VMEM((CHUNK,), f32),
            rows2=pltpu.MemorySpace.VMEM((CHUNK,), i32),
            cols2=pltpu.MemorySpace.VMEM((CHUNK,), i32),
            data2=pltpu.MemorySpace.VMEM((CHUNK,), f32),
            tmp_v=pltpu.MemorySpace.VMEM((CHUNK,), f32),
            tmp2_v=pltpu.MemorySpace.VMEM((CHUNK,), f32),
            semr0=pltpu.SemaphoreType.DMA,
            semc0=pltpu.SemaphoreType.DMA,
            semd0=pltpu.SemaphoreType.DMA,
            semr1=pltpu.SemaphoreType.DMA,
            semc1=pltpu.SemaphoreType.DMA,
            semd1=pltpu.SemaphoreType.DMA,
            semr2=pltpu.SemaphoreType.DMA,
            semc2=pltpu.SemaphoreType.DMA,
            semd2=pltpu.SemaphoreType.DMA,
            semx=pltpu.SemaphoreType.DMA,
            semy=pltpu.SemaphoreType.DMA,
            semz=pltpu.SemaphoreType.DMA,
        ),
    )
    def spmv(p_data, p_rows, p_cols, a_data, a_rows, a_cols, x_h, y_h, s_h,
             o_px, o_aty, o_ax,
             x_v, wy_v, acc_px, acc_aty, acc_ax,
             rows0, cols0, data0, rows1, cols1, data1,
             rows2, cols2, data2, tmp_v, tmp2_v,
             semr0, semc0, semd0, semr1, semc1, semd1,
             semr2, semc2, semd2, semx, semy, semz):
        cid = lax.axis_index("c")
        sid = lax.axis_index("s")
        wid = cid * NS + sid
        base = wid * nnz_per_w

        sets = ((rows0, cols0, data0, semr0, semc0, semd0),
                (rows1, cols1, data1, semr1, semc1, semd1),
                (rows2, cols2, data2, semr2, semc2, semd2))
        nbuf = len(sets)

        def issue(buf, dh, rh, ch_, off):
            rows_v, cols_v, data_v, sr, sc, sd = buf
            c0 = pltpu.async_copy(rh.at[pl.ds(off, CHUNK)], rows_v, sr)
            c1 = pltpu.async_copy(ch_.at[pl.ds(off, CHUNK)], cols_v, sc)
            c2 = pltpu.async_copy(dh.at[pl.ds(off, CHUNK)], data_v, sd)
            return c0, c1, c2

        def wait(buf, dh, rh, ch_):
            rows_v, cols_v, data_v, sr, sc, sd = buf
            pltpu.make_async_copy(rh.at[pl.ds(0, CHUNK)], rows_v, sr).wait()
            pltpu.make_async_copy(ch_.at[pl.ds(0, CHUNK)], cols_v, sc).wait()
            pltpu.make_async_copy(dh.at[pl.ds(0, CHUNK)], data_v, sd).wait()

        def process_p(buf):
            rows_v, cols_v, data_v, *_ = buf

            @plsc.parallel_loop(0, VPC, unroll=16)
            def _(i):
                sl = pl.ds(i * LANES, LANES)
                vals = data_v[sl] * plsc.load_gather(x_v, [cols_v[sl]])
                plsc.addupdate_scatter(acc_px, [rows_v[sl]], vals)

        def process_a(buf):
            rows_v, cols_v, data_v, *_ = buf

            @plsc.parallel_loop(0, VPC, unroll=16)
            def _(i):
                sl = pl.ds(i * LANES, LANES)
                rows = rows_v[sl]
                cols = cols_v[sl]
                data = data_v[sl]
                plsc.addupdate_scatter(acc_ax, [rows],
                                       data * plsc.load_gather(x_v, [cols]))
                plsc.addupdate_scatter(acc_aty, [cols],
                                       data * plsc.load_gather(wy_v, [rows]))

        def pass_over(dh, rh, ch_, process, prefetched=False):
            n_groups = -(-n_chunks // nbuf)
            if not prefetched:
                for k in range(nbuf):
                    issue(sets[k], dh, rh, ch_, base + k * CHUNK)

            def group_body(g, _):
                for k in range(nbuf):
                    ch = g * nbuf + k

                    @pl.when(ch < n_chunks)
                    def _():
                        wait(sets[k], dh, rh, ch_)
                        process(sets[k])

                    @pl.when(ch + nbuf < n_chunks)
                    def _():
                        issue(sets[k], dh, rh, ch_,
                              base + (ch + nbuf) * CHUNK)

                return 0

            lax.fori_loop(0, n_groups, group_body, 0)

        for k in range(nbuf):
            issue(sets[k], p_data, p_rows, p_cols, base + k * CHUNK)
        cx = pltpu.async_copy(x_h, x_v, semx)
        for ch in range(m // CHUNK):
            off = ch * CHUNK
            cy = pltpu.async_copy(y_h.at[pl.ds(off, CHUNK)], tmp_v, semy)
            cs = pltpu.async_copy(s_h.at[pl.ds(off, CHUNK)], tmp2_v, semz)
            cy.wait()
            cs.wait()

            @plsc.parallel_loop(0, VPC, unroll=8)
            def _(i):
                z = jnp.zeros((LANES,), f32)
                sl = pl.ds(i * LANES, LANES)
                gsl = pl.ds(off + i * LANES, LANES)
                wy_v[gsl] = jnp.maximum(tmp_v[sl] - tmp2_v[sl], 0.0)
                acc_px[gsl] = z
                acc_aty[gsl] = z
                acc_ax[gsl] = z

        cx.wait()

        pass_over(p_data, p_rows, p_cols, process_p, prefetched=True)
        pass_over(a_data, a_rows, a_cols, process_a)

        e0 = pltpu.async_copy(acc_px, o_px.at[wid], semx)
        e1 = pltpu.async_copy(acc_aty, o_aty.at[wid], semy)
        e2 = pltpu.async_copy(acc_ax, o_ax.at[wid], semz)
        e0.wait()
        e1.wait()
        e2.wait()

    return spmv


def _tc_combine(p_px, p_aty, p_ax, q2, b2, x2, y2, s2):
    n = q2.shape[1]
    m = b2.shape[1]
    f32 = jnp.float32

    def body(px_ref, aty_ref, ax_ref, q_ref, b_ref, x_ref, y_ref, s_ref,
             out_ref):
        px = jnp.sum(px_ref[...], axis=0, keepdims=True)
        aty = jnp.sum(aty_ref[...], axis=0, keepdims=True)
        ax = jnp.sum(ax_ref[...], axis=0, keepdims=True)
        q = q_ref[...]
        b = b_ref[...]
        x = x_ref[...]
        wy = jnp.maximum(y_ref[...] - s_ref[...], 0.0)
        out_ref[:, pl.ds(0, n)] = px + aty + q
        out_ref[:, pl.ds(n, m)] = b - ax
        xtpx = jnp.sum(x * px)
        qx = jnp.sum(q * x)
        bwy = jnp.sum(b * wy)
        out_ref[:, pl.ds(n + m, 1)] = jnp.reshape(
            -(qx + xtpx) - bwy + 0.5 * xtpx, (1, 1))

    return pl.pallas_call(
        body,
        out_shape=jax.ShapeDtypeStruct((1, n + m + 1), f32),
    )(p_px, p_aty, p_ax, q2, b2, x2, y2, s2)


def kernel(P_data, A_data, q, b, x, y, s, P_rows, P_cols, A_rows, A_cols):
    n = x.shape[0]
    m = y.shape[0]
    nnz = P_data.shape[0]

    spmv = _sc_spmv_kernel(n, m, nnz)
    p_px, p_aty, p_ax = spmv(P_data, P_rows, P_cols, A_data, A_rows, A_cols,
                             x, y, s)

    out = _tc_combine(
        p_px, p_aty, p_ax,
        q.reshape(1, n), b.reshape(1, m), x.reshape(1, n),
        y.reshape(1, m), s.reshape(1, m))

    return out.reshape(-1)


# speedup vs baseline: 1.0335x; 1.0216x over previous
import functools

import jax
import jax.numpy as jnp
from jax import lax
from jax.experimental import pallas as pl
from jax.experimental.pallas import tpu as pltpu
from jax.experimental.pallas import tpu_sc as plsc

NC = 2
NS = 16
NW = NC * NS
LANES = 16
CHUNK = 4096
VPC = CHUNK // LANES


def _sc_spmv_kernel(n, m, nnz):
    nnz_per_w = nnz // NW
    n_chunks = nnz_per_w // CHUNK
    assert nnz_per_w % CHUNK == 0 and n_chunks % 2 == 0
    assert n == m and n % CHUNK == 0

    mesh = plsc.VectorSubcoreMesh(core_axis_name="c", subcore_axis_name="s")
    f32 = jnp.float32
    i32 = jnp.int32

    @functools.partial(
        pl.kernel,
        out_type=(
            jax.ShapeDtypeStruct((NW, n), f32),
            jax.ShapeDtypeStruct((NW, n), f32),
            jax.ShapeDtypeStruct((NW, m), f32),
        ),
        mesh=mesh,
        compiler_params=pltpu.CompilerParams(needs_layout_passes=False),
        scratch_types=dict(
            x_v=pltpu.MemorySpace.VMEM((n,), f32),
            wy_v=pltpu.MemorySpace.VMEM((m,), f32),
            acc_px=pltpu.MemorySpace.VMEM((n,), f32),
            acc_aty=pltpu.MemorySpace.VMEM((n,), f32),
            acc_ax=pltpu.MemorySpace.VMEM((m,), f32),
            rows0=pltpu.MemorySpace.VMEM((CHUNK,), i32),
            cols0=pltpu.MemorySpace.VMEM((CHUNK,), i32),
            data0=pltpu.MemorySpace.VMEM((CHUNK,), f32),
            rows1=pltpu.MemorySpace.VMEM((CHUNK,), i32),
            cols1=pltpu.MemorySpace.VMEM((CHUNK,), i32),
            data1=pltpu.MemorySpace.VMEM((CHUNK,), f32),
            rows2=pltpu.MemorySpace.VMEM((CHUNK,), i32),
            cols2=pltpu.MemorySpace.VMEM((CHUNK,), i32),
            data2=pltpu.MemorySpace.VMEM((CHUNK,), f32),
            tmp_v=pltpu.MemorySpace.VMEM((CHUNK,), f32),
            tmp2_v=pltpu.MemorySpace.VMEM((CHUNK,), f32),
            semr0=pltpu.SemaphoreType.DMA,
            semc0=pltpu.SemaphoreType.DMA,
            semd0=pltpu.SemaphoreType.DMA,
            semr1=pltpu.SemaphoreType.DMA,
            semc1=pltpu.SemaphoreType.DMA,
            semd1=pltpu.SemaphoreType.DMA,
            semr2=pltpu.SemaphoreType.DMA,
            semc2=pltpu.SemaphoreType.DMA,
            semd2=pltpu.SemaphoreType.DMA,
            semx=pltpu.SemaphoreType.DMA,
            semy=pltpu.SemaphoreType.DMA,
            semz=pltpu.SemaphoreType.DMA,
        ),
    )
    def spmv(p_data, p_rows, p_cols, a_data, a_rows, a_cols, x_h, y_h, s_h,
             o_px, o_aty, o_ax,
             x_v, wy_v, acc_px, acc_aty, acc_ax,
             rows0, cols0, data0, rows1, cols1, data1,
             rows2, cols2, data2, tmp_v, tmp2_v,
             semr0, semc0, semd0, semr1, semc1, semd1,
             semr2, semc2, semd2, semx, semy, semz):
        cid = lax.axis_index("c")
        sid = lax.axis_index("s")
        wid = cid * NS + sid
        base = wid * nnz_per_w

        sets = ((rows0, cols0, data0, semr0, semc0, semd0),
                (rows1, cols1, data1, semr1, semc1, semd1),
                (rows2, cols2, data2, semr2, semc2, semd2))
        nbuf = len(sets)

        def issue(buf, dh, rh, ch_, off):
            rows_v, cols_v, data_v, sr, sc, sd = buf
            c0 = pltpu.async_copy(rh.at[pl.ds(off, CHUNK)], rows_v, sr)
            c1 = pltpu.async_copy(ch_.at[pl.ds(off, CHUNK)], cols_v, sc)
            c2 = pltpu.async_copy(dh.at[pl.ds(off, CHUNK)], data_v, sd)
            return c0, c1, c2

        def wait(buf, dh, rh, ch_):
            rows_v, cols_v, data_v, sr, sc, sd = buf
            pltpu.make_async_copy(rh.at[pl.ds(0, CHUNK)], rows_v, sr).wait()
            pltpu.make_async_copy(ch_.at[pl.ds(0, CHUNK)], cols_v, sc).wait()
            pltpu.make_async_copy(dh.at[pl.ds(0, CHUNK)], data_v, sd).wait()

        def process_p(buf):
            rows_v, cols_v, data_v, *_ = buf

            @plsc.parallel_loop(0, VPC, unroll=16)
            def _(i):
                sl = pl.ds(i * LANES, LANES)
                vals = data_v[sl] * plsc.load_gather(x_v, [cols_v[sl]])
                plsc.addupdate_scatter(acc_px, [rows_v[sl]], vals)

        def process_a(buf):
            rows_v, cols_v, data_v, *_ = buf

            @plsc.parallel_loop(0, VPC, unroll=16)
            def _(i):
                sl = pl.ds(i * LANES, LANES)
                rows = rows_v[sl]
                cols = cols_v[sl]
                data = data_v[sl]
                plsc.addupdate_scatter(acc_ax, [rows],
                                       data * plsc.load_gather(x_v, [cols]))
                wyg = plsc.load_gather(wy_v, [rows])
                plsc.addupdate_scatter(acc_aty, [cols], data * wyg,
                                       mask=wyg > 0.0)

        def pass_over(dh, rh, ch_, process, prefetched=False):
            n_groups = -(-n_chunks // nbuf)
            if not prefetched:
                for k in range(nbuf):
                    issue(sets[k], dh, rh, ch_, base + k * CHUNK)

            def group_body(g, _):
                for k in range(nbuf):
                    ch = g * nbuf + k

                    @pl.when(ch < n_chunks)
                    def _():
                        wait(sets[k], dh, rh, ch_)
                        process(sets[k])

                    @pl.when(ch + nbuf < n_chunks)
                    def _():
                        issue(sets[k], dh, rh, ch_,
                              base + (ch + nbuf) * CHUNK)

                return 0

            lax.fori_loop(0, n_groups, group_body, 0)

        for k in range(nbuf):
            issue(sets[k], p_data, p_rows, p_cols, base + k * CHUNK)
        cx = pltpu.async_copy(x_h, x_v, semx)
        for ch in range(m // CHUNK):
            off = ch * CHUNK
            cy = pltpu.async_copy(y_h.at[pl.ds(off, CHUNK)], tmp_v, semy)
            cs = pltpu.async_copy(s_h.at[pl.ds(off, CHUNK)], tmp2_v, semz)
            cy.wait()
            cs.wait()

            @plsc.parallel_loop(0, VPC, unroll=8)
            def _(i):
                z = jnp.zeros((LANES,), f32)
                sl = pl.ds(i * LANES, LANES)
                gsl = pl.ds(off + i * LANES, LANES)
                wy_v[gsl] = jnp.maximum(tmp_v[sl] - tmp2_v[sl], 0.0)
                acc_px[gsl] = z
                acc_aty[gsl] = z
                acc_ax[gsl] = z

        cx.wait()

        pass_over(p_data, p_rows, p_cols, process_p, prefetched=True)
        pass_over(a_data, a_rows, a_cols, process_a)

        e0 = pltpu.async_copy(acc_px, o_px.at[wid], semx)
        e1 = pltpu.async_copy(acc_aty, o_aty.at[wid], semy)
        e2 = pltpu.async_copy(acc_ax, o_ax.at[wid], semz)
        e0.wait()
        e1.wait()
        e2.wait()

    return spmv


def _tc_combine(p_px, p_aty, p_ax, q2, b2, x2, y2, s2):
    n = q2.shape[1]
    m = b2.shape[1]
    f32 = jnp.float32

    def body(px_ref, aty_ref, ax_ref, q_ref, b_ref, x_ref, y_ref, s_ref,
             out_ref):
        px = jnp.sum(px_ref[...], axis=0, keepdims=True)
        aty = jnp.sum(aty_ref[...], axis=0, keepdims=True)
        ax = jnp.sum(ax_ref[...], axis=0, keepdims=True)
        q = q_ref[...]
        b = b_ref[...]
        x = x_ref[...]
        wy = jnp.maximum(y_ref[...] - s_ref[...], 0.0)
        out_ref[:, pl.ds(0, n)] = px + aty + q
        out_ref[:, pl.ds(n, m)] = b - ax
        xtpx = jnp.sum(x * px)
        qx = jnp.sum(q * x)
        bwy = jnp.sum(b * wy)
        out_ref[:, pl.ds(n + m, 1)] = jnp.reshape(
            -(qx + xtpx) - bwy + 0.5 * xtpx, (1, 1))

    return pl.pallas_call(
        body,
        out_shape=jax.ShapeDtypeStruct((1, n + m + 1), f32),
    )(p_px, p_aty, p_ax, q2, b2, x2, y2, s2)


def kernel(P_data, A_data, q, b, x, y, s, P_rows, P_cols, A_rows, A_cols):
    n = x.shape[0]
    m = y.shape[0]
    nnz = P_data.shape[0]

    spmv = _sc_spmv_kernel(n, m, nnz)
    p_px, p_aty, p_ax = spmv(P_data, P_rows, P_cols, A_data, A_rows, A_cols,
                             x, y, s)

    out = _tc_combine(
        p_px, p_aty, p_ax,
        q.reshape(1, n), b.reshape(1, m), x.reshape(1, n),
        y.reshape(1, m), s.reshape(1, m))

    return out.reshape(-1)


# speedup vs baseline: 1.0513x; 1.0172x over previous
import functools

import jax
import jax.numpy as jnp
from jax import lax
from jax.experimental import pallas as pl
from jax.experimental.pallas import tpu as pltpu
from jax.experimental.pallas import tpu_sc as plsc

NC = 2
NS = 16
NW = NC * NS
LANES = 16
CHUNK = 4096
VPC = CHUNK // LANES


def _sc_spmv_kernel(n, m, nnz):
    nnz_per_w = nnz // NW
    n_chunks = nnz_per_w // CHUNK
    assert nnz_per_w % CHUNK == 0 and n_chunks % 2 == 0
    assert n == m and n % CHUNK == 0

    mesh = plsc.VectorSubcoreMesh(core_axis_name="c", subcore_axis_name="s")
    f32 = jnp.float32
    i32 = jnp.int32

    @functools.partial(
        pl.kernel,
        out_type=(
            jax.ShapeDtypeStruct((NW, n), f32),
            jax.ShapeDtypeStruct((NW, n), f32),
            jax.ShapeDtypeStruct((NW, m), f32),
        ),
        mesh=mesh,
        compiler_params=pltpu.CompilerParams(needs_layout_passes=False),
        scratch_types=dict(
            x_v=pltpu.MemorySpace.VMEM((n,), f32),
            wy_v=pltpu.MemorySpace.VMEM((m,), f32),
            acc_px=pltpu.MemorySpace.VMEM((n,), f32),
            acc_aty=pltpu.MemorySpace.VMEM((n,), f32),
            acc_ax=pltpu.MemorySpace.VMEM((m,), f32),
            rows0=pltpu.MemorySpace.VMEM((CHUNK,), i32),
            cols0=pltpu.MemorySpace.VMEM((CHUNK,), i32),
            data0=pltpu.MemorySpace.VMEM((CHUNK,), f32),
            rows1=pltpu.MemorySpace.VMEM((CHUNK,), i32),
            cols1=pltpu.MemorySpace.VMEM((CHUNK,), i32),
            data1=pltpu.MemorySpace.VMEM((CHUNK,), f32),
            rows2=pltpu.MemorySpace.VMEM((CHUNK,), i32),
            cols2=pltpu.MemorySpace.VMEM((CHUNK,), i32),
            data2=pltpu.MemorySpace.VMEM((CHUNK,), f32),
            tmp_v=pltpu.MemorySpace.VMEM((CHUNK,), f32),
            tmp2_v=pltpu.MemorySpace.VMEM((CHUNK,), f32),
            semr0=pltpu.SemaphoreType.DMA,
            semc0=pltpu.SemaphoreType.DMA,
            semd0=pltpu.SemaphoreType.DMA,
            semr1=pltpu.SemaphoreType.DMA,
            semc1=pltpu.SemaphoreType.DMA,
            semd1=pltpu.SemaphoreType.DMA,
            semr2=pltpu.SemaphoreType.DMA,
            semc2=pltpu.SemaphoreType.DMA,
            semd2=pltpu.SemaphoreType.DMA,
            semx=pltpu.SemaphoreType.DMA,
            semy=pltpu.SemaphoreType.DMA,
            semz=pltpu.SemaphoreType.DMA,
        ),
    )
    def spmv(p_data, p_rows, p_cols, a_data, a_rows, a_cols, x_h, y_h, s_h,
             o_px, o_aty, o_ax,
             x_v, wy_v, acc_px, acc_aty, acc_ax,
             rows0, cols0, data0, rows1, cols1, data1,
             rows2, cols2, data2, tmp_v, tmp2_v,
             semr0, semc0, semd0, semr1, semc1, semd1,
             semr2, semc2, semd2, semx, semy, semz):
        cid = lax.axis_index("c")
        sid = lax.axis_index("s")
        wid = cid * NS + sid
        base = wid * nnz_per_w

        sets = ((rows0, cols0, data0, semr0, semc0, semd0),
                (rows1, cols1, data1, semr1, semc1, semd1),
                (rows2, cols2, data2, semr2, semc2, semd2))
        nbuf = len(sets)

        def issue(buf, dh, rh, ch_, off):
            rows_v, cols_v, data_v, sr, sc, sd = buf
            c0 = pltpu.async_copy(rh.at[pl.ds(off, CHUNK)], rows_v, sr)
            c1 = pltpu.async_copy(ch_.at[pl.ds(off, CHUNK)], cols_v, sc)
            c2 = pltpu.async_copy(dh.at[pl.ds(off, CHUNK)], data_v, sd)
            return c0, c1, c2

        def wait(buf, dh, rh, ch_):
            rows_v, cols_v, data_v, sr, sc, sd = buf
            pltpu.make_async_copy(rh.at[pl.ds(0, CHUNK)], rows_v, sr).wait()
            pltpu.make_async_copy(ch_.at[pl.ds(0, CHUNK)], cols_v, sc).wait()
            pltpu.make_async_copy(dh.at[pl.ds(0, CHUNK)], data_v, sd).wait()

        def process_p(buf):
            rows_v, cols_v, data_v, *_ = buf

            @plsc.parallel_loop(0, VPC, unroll=16)
            def _(i):
                sl = pl.ds(i * LANES, LANES)
                vals = data_v[sl] * plsc.load_gather(x_v, [cols_v[sl]])
                plsc.addupdate_scatter(acc_px, [rows_v[sl]], vals)

        def process_a(buf):
            rows_v, cols_v, data_v, *_ = buf

            @plsc.parallel_loop(0, VPC, unroll=16)
            def _(i):
                sl = pl.ds(i * LANES, LANES)
                rows = rows_v[sl]
                cols = cols_v[sl]
                data = data_v[sl]
                plsc.addupdate_scatter(acc_ax, [rows],
                                       data * plsc.load_gather(x_v, [cols]))
                wyg = plsc.load_gather(wy_v, [rows])
                plsc.addupdate_scatter(acc_aty, [cols], data * wyg,
                                       mask=wyg > 0.0)

        def pass_over(dh, rh, ch_, process, prefetched=False):
            if not prefetched:
                issue(sets[0], dh, rh, ch_, base)

            def pair_body(p, _):
                off0 = base + (2 * p) * CHUNK
                wait(sets[0], dh, rh, ch_)
                issue(sets[1], dh, rh, ch_, off0 + CHUNK)
                process(sets[0])
                wait(sets[1], dh, rh, ch_)

                @pl.when(2 * p + 2 < n_chunks)
                def _():
                    issue(sets[0], dh, rh, ch_, off0 + 2 * CHUNK)

                process(sets[1])
                return 0

            lax.fori_loop(0, n_chunks // 2, pair_body, 0)

        issue(sets[0], p_data, p_rows, p_cols, base)
        cx = pltpu.async_copy(x_h, x_v, semx)
        for ch in range(m // CHUNK):
            off = ch * CHUNK
            cy = pltpu.async_copy(y_h.at[pl.ds(off, CHUNK)], tmp_v, semy)
            cs = pltpu.async_copy(s_h.at[pl.ds(off, CHUNK)], tmp2_v, semz)
            cy.wait()
            cs.wait()

            @plsc.parallel_loop(0, VPC, unroll=8)
            def _(i):
                z = jnp.zeros((LANES,), f32)
                sl = pl.ds(i * LANES, LANES)
                gsl = pl.ds(off + i * LANES, LANES)
                wy_v[gsl] = jnp.maximum(tmp_v[sl] - tmp2_v[sl], 0.0)
                acc_px[gsl] = z
                acc_aty[gsl] = z
                acc_ax[gsl] = z

        cx.wait()

        pass_over(p_data, p_rows, p_cols, process_p, prefetched=True)
        pass_over(a_data, a_rows, a_cols, process_a)

        e0 = pltpu.async_copy(acc_px, o_px.at[wid], semx)
        e1 = pltpu.async_copy(acc_aty, o_aty.at[wid], semy)
        e2 = pltpu.async_copy(acc_ax, o_ax.at[wid], semz)
        e0.wait()
        e1.wait()
        e2.wait()

    return spmv


def _tc_combine(p_px, p_aty, p_ax, q2, b2, x2, y2, s2):
    n = q2.shape[1]
    m = b2.shape[1]
    f32 = jnp.float32

    def body(px_ref, aty_ref, ax_ref, q_ref, b_ref, x_ref, y_ref, s_ref,
             out_ref):
        px = jnp.sum(px_ref[...], axis=0, keepdims=True)
        aty = jnp.sum(aty_ref[...], axis=0, keepdims=True)
        ax = jnp.sum(ax_ref[...], axis=0, keepdims=True)
        q = q_ref[...]
        b = b_ref[...]
        x = x_ref[...]
        wy = jnp.maximum(y_ref[...] - s_ref[...], 0.0)
        out_ref[:, pl.ds(0, n)] = px + aty + q
        out_ref[:, pl.ds(n, m)] = b - ax
        xtpx = jnp.sum(x * px)
        qx = jnp.sum(q * x)
        bwy = jnp.sum(b * wy)
        out_ref[:, pl.ds(n + m, 1)] = jnp.reshape(
            -(qx + xtpx) - bwy + 0.5 * xtpx, (1, 1))

    return pl.pallas_call(
        body,
        out_shape=jax.ShapeDtypeStruct((1, n + m + 1), f32),
    )(p_px, p_aty, p_ax, q2, b2, x2, y2, s2)


def kernel(P_data, A_data, q, b, x, y, s, P_rows, P_cols, A_rows, A_cols):
    n = x.shape[0]
    m = y.shape[0]
    nnz = P_data.shape[0]

    spmv = _sc_spmv_kernel(n, m, nnz)
    p_px, p_aty, p_ax = spmv(P_data, P_rows, P_cols, A_data, A_rows, A_cols,
                             x, y, s)

    out = _tc_combine(
        p_px, p_aty, p_ax,
        q.reshape(1, n), b.reshape(1, m), x.reshape(1, n),
        y.reshape(1, m), s.reshape(1, m))

    return out.reshape(-1)
